# baseline (device time: 53573 ns/iter reference)
import jax
import jax.numpy as jnp
from jax import lax
from jax.experimental import pallas as pl
from jax.experimental.pallas import tpu as pltpu

N_DEV = 4
B, Sq, Skv, HQ_TOTAL, Dh = 2, 512, 512, 32, 64
HQ_LOCAL = HQ_TOTAL // N_DEV
D_MODEL = 768
BLK = 64
D_LOCAL = HQ_LOCAL * Dh

ROWS = B * Sq
N_CHUNK = 2 * N_DEV
CHUNK = ROWS // N_CHUNK
N_STEP = 2 * (N_DEV - 1)

bf16 = jnp.bfloat16
f32 = jnp.float32


def _m4(e):
    return lax.rem(e, N_DEV)


def _fused_body(
    x_ref, wq_ref, kt_ref, v_ref, wo_ref,
    out_ref,
    xb, qbuf, stage_ref, recv_ref, send_sems, recv_sems,
):
    my = lax.axis_index("i")
    left = _m4(my + N_DEV - 1)
    right = _m4(my + 1)

    xb[...] = x_ref[...].astype(bf16)
    for h in range(HQ_LOCAL):
        qbuf[h] = jnp.dot(
            xb[...], wq_ref[h], preferred_element_type=f32
        ).astype(bf16)

    qb = lax.broadcasted_iota(jnp.int32, (Sq, Skv), 0) // BLK
    kb = lax.broadcasted_iota(jnp.int32, (Sq, Skv), 1) // BLK
    mask = (qb == kb) | (kb == 0) | (lax.rem(qb + kb, 3) == 0)

    out_ref[...] = jnp.zeros((ROWS, D_MODEL), f32)
    for b in range(B):
        for h in range(HQ_LOCAL):
            bh = b * HQ_LOCAL + h
            q = qbuf[h, b * Sq:(b + 1) * Sq, :]
            s = jnp.dot(q, kt_ref[bh], preferred_element_type=f32) * 0.125
            s = jnp.where(mask, s, -1e9)
            m = jnp.max(s, axis=1, keepdims=True)
            e = jnp.exp(s - m)
            w = (e / jnp.sum(e, axis=1, keepdims=True)).astype(bf16)
            ctx = jnp.dot(w, v_ref[bh], preferred_element_type=f32)
            out_ref[b * Sq:(b + 1) * Sq, :] += jnp.dot(
                ctx.astype(bf16), wo_ref[h], preferred_element_type=f32
            )

    barrier_sem = pltpu.get_barrier_semaphore()
    for nbr in (left, right):
        pl.semaphore_signal(
            barrier_sem, inc=1,
            device_id=(nbr,), device_id_type=pl.DeviceIdType.MESH,
        )
    pl.semaphore_wait(barrier_sem, 2)

    def rows(c):
        return pl.ds(c * CHUNK, CHUNK)

    for s in range(N_DEV - 1):
        plan = [
            (0, right, _m4(my + N_DEV - s), _m4(my + N_DEV - s - 1)),
            (1, left, N_DEV + _m4(my + s), N_DEV + _m4(my + s + 1)),
        ]
        rdmas = []
        for d, nbr, c_send, _ in plan:
            stage_ref[d] = out_ref[rows(c_send), :].astype(bf16)
            rdma = pltpu.make_async_remote_copy(
                src_ref=stage_ref.at[d],
                dst_ref=recv_ref.at[d, s],
                send_sem=send_sems.at[d, s],
                recv_sem=recv_sems.at[d, s],
                device_id=(nbr,),
                device_id_type=pl.DeviceIdType.MESH,
            )
            rdma.start()
            rdmas.append(rdma)
        for (d, _, _, c_recv), rdma in zip(plan, rdmas):
            rdma.wait()
            out_ref[rows(c_recv), :] += recv_ref[d, s].astype(f32)

    for s in range(N_DEV - 1):
        t = N_DEV - 1 + s
        plan = [
            (0, right, _m4(my + N_DEV + 1 - s), _m4(my + N_DEV - s)),
            (1, left, N_DEV + _m4(my + N_DEV - 1 + s), N_DEV + _m4(my + s)),
        ]
        rdmas = []
        for d, nbr, c_send, _ in plan:
            stage_ref[d] = out_ref[rows(c_send), :].astype(bf16)
            rdma = pltpu.make_async_remote_copy(
                src_ref=stage_ref.at[d],
                dst_ref=recv_ref.at[d, t],
                send_sem=send_sems.at[d, t],
                recv_sem=recv_sems.at[d, t],
                device_id=(nbr,),
                device_id_type=pl.DeviceIdType.MESH,
            )
            rdma.start()
            rdmas.append(rdma)
        for (d, _, _, c_recv), rdma in zip(plan, rdmas):
            rdma.wait()
            out_ref[rows(c_recv), :] = recv_ref[d, t].astype(f32)


def kernel(x, Wq, K_ext, V_ext, Wo):
    my = lax.axis_index("i")

    Kl = lax.dynamic_slice_in_dim(K_ext, my * HQ_LOCAL, HQ_LOCAL, axis=2)
    KT = Kl.transpose(0, 2, 3, 1).reshape(B * HQ_LOCAL, Dh, Skv).astype(bf16)
    Vl = (
        lax.dynamic_slice_in_dim(V_ext, my * HQ_LOCAL, HQ_LOCAL, axis=2)
        .transpose(0, 2, 1, 3)
        .reshape(B * HQ_LOCAL, Skv, Dh)
        .astype(bf16)
    )
    WqT = Wq.reshape(D_MODEL, HQ_LOCAL, Dh).transpose(1, 0, 2).astype(bf16)
    Wo3 = Wo.reshape(HQ_LOCAL, Dh, D_MODEL).astype(bf16)

    out = pl.pallas_call(
        _fused_body,
        out_shape=jax.ShapeDtypeStruct((ROWS, D_MODEL), f32),
        in_specs=[pl.BlockSpec(memory_space=pltpu.VMEM)] * 5,
        out_specs=pl.BlockSpec(memory_space=pltpu.VMEM),
        scratch_shapes=[
            pltpu.VMEM((ROWS, D_MODEL), bf16),
            pltpu.VMEM((HQ_LOCAL, ROWS, Dh), bf16),
            pltpu.VMEM((2, CHUNK, D_MODEL), bf16),
            pltpu.VMEM((2, N_STEP, CHUNK, D_MODEL), bf16),
            pltpu.SemaphoreType.DMA((2, N_STEP)),
            pltpu.SemaphoreType.DMA((2, N_STEP)),
        ],
        compiler_params=pltpu.CompilerParams(collective_id=0),
    )(x.reshape(ROWS, D_MODEL), WqT, KT, Vl, Wo3)
    return out.reshape(B, Sq, D_MODEL)


# device time: 50329 ns/iter; 1.0645x vs baseline; 1.0645x over previous
import jax
import jax.numpy as jnp
from jax import lax
from jax.experimental import pallas as pl
from jax.experimental.pallas import tpu as pltpu

N_DEV = 4
B, Sq, Skv, HQ_TOTAL, Dh = 2, 512, 512, 32, 64
HQ_LOCAL = HQ_TOTAL // N_DEV
D_MODEL = 768
BLK = 64
D_LOCAL = HQ_LOCAL * Dh

ROWS = B * Sq
N_CHUNK = 2 * N_DEV
CHUNK = ROWS // N_CHUNK
N_STEP = 2 * (N_DEV - 1)

bf16 = jnp.bfloat16
f32 = jnp.float32


def _m4(e):
    return lax.rem(e, N_DEV)


def _fused_body(
    x_ref, wq_ref, kt_ref, v_ref, wo_ref,
    out_ref,
    xb, qbuf, stage_ref, recv_ref, send_sems, recv_sems,
):
    my = lax.axis_index("i")
    left = _m4(my + N_DEV - 1)
    right = _m4(my + 1)

    xb[...] = x_ref[...].astype(bf16)
    for h in range(HQ_LOCAL):
        qbuf[h] = (
            jnp.dot(xb[...], wq_ref[h], preferred_element_type=f32) * 0.125
        ).astype(bf16)

    qb = lax.broadcasted_iota(jnp.int32, (Sq, Skv), 0) // BLK
    kb = lax.broadcasted_iota(jnp.int32, (Sq, Skv), 1) // BLK
    mask = (qb == kb) | (kb == 0) | (lax.rem(qb + kb, 3) == 0)
    bias = jnp.where(mask, 0.0, -1e9).astype(f32)

    out_ref[...] = jnp.zeros((ROWS, D_MODEL), f32)
    for b in range(B):
        for h in range(HQ_LOCAL):
            bh = b * HQ_LOCAL + h
            q = qbuf[h, b * Sq:(b + 1) * Sq, :]
            s = jnp.dot(q, kt_ref[bh], preferred_element_type=f32) + bias
            e = jnp.exp(s)
            rs = 1.0 / jnp.sum(e, axis=1, keepdims=True)
            w = (e * rs).astype(bf16)
            ctx = jnp.dot(w, v_ref[bh], preferred_element_type=f32)
            out_ref[b * Sq:(b + 1) * Sq, :] += jnp.dot(
                ctx.astype(bf16), wo_ref[h], preferred_element_type=f32
            )

    barrier_sem = pltpu.get_barrier_semaphore()
    for nbr in (left, right):
        pl.semaphore_signal(
            barrier_sem, inc=1,
            device_id=(nbr,), device_id_type=pl.DeviceIdType.MESH,
        )
    pl.semaphore_wait(barrier_sem, 2)

    def rows(c):
        return pl.ds(c * CHUNK, CHUNK)

    for s in range(N_DEV - 1):
        plan = [
            (0, right, _m4(my + N_DEV - s), _m4(my + N_DEV - s - 1)),
            (1, left, N_DEV + _m4(my + s), N_DEV + _m4(my + s + 1)),
        ]
        rdmas = []
        for d, nbr, c_send, _ in plan:
            stage_ref[d] = out_ref[rows(c_send), :].astype(bf16)
            rdma = pltpu.make_async_remote_copy(
                src_ref=stage_ref.at[d],
                dst_ref=recv_ref.at[d, s],
                send_sem=send_sems.at[d, s],
                recv_sem=recv_sems.at[d, s],
                device_id=(nbr,),
                device_id_type=pl.DeviceIdType.MESH,
            )
            rdma.start()
            rdmas.append(rdma)
        for (d, _, _, c_recv), rdma in zip(plan, rdmas):
            rdma.wait()
            out_ref[rows(c_recv), :] += recv_ref[d, s].astype(f32)

    for s in range(N_DEV - 1):
        t = N_DEV - 1 + s
        plan = [
            (0, right, _m4(my + N_DEV + 1 - s), _m4(my + N_DEV - s)),
            (1, left, N_DEV + _m4(my + N_DEV - 1 + s), N_DEV + _m4(my + s)),
        ]
        rdmas = []
        for d, nbr, c_send, _ in plan:
            stage_ref[d] = out_ref[rows(c_send), :].astype(bf16)
            rdma = pltpu.make_async_remote_copy(
                src_ref=stage_ref.at[d],
                dst_ref=recv_ref.at[d, t],
                send_sem=send_sems.at[d, t],
                recv_sem=recv_sems.at[d, t],
                device_id=(nbr,),
                device_id_type=pl.DeviceIdType.MESH,
            )
            rdma.start()
            rdmas.append(rdma)
        for (d, _, _, c_recv), rdma in zip(plan, rdmas):
            rdma.wait()
            out_ref[rows(c_recv), :] = recv_ref[d, t].astype(f32)


def kernel(x, Wq, K_ext, V_ext, Wo):
    my = lax.axis_index("i")

    Kl = lax.dynamic_slice_in_dim(K_ext, my * HQ_LOCAL, HQ_LOCAL, axis=2)
    KT = Kl.transpose(0, 2, 3, 1).reshape(B * HQ_LOCAL, Dh, Skv).astype(bf16)
    Vl = (
        lax.dynamic_slice_in_dim(V_ext, my * HQ_LOCAL, HQ_LOCAL, axis=2)
        .transpose(0, 2, 1, 3)
        .reshape(B * HQ_LOCAL, Skv, Dh)
        .astype(bf16)
    )
    WqT = Wq.reshape(D_MODEL, HQ_LOCAL, Dh).transpose(1, 0, 2).astype(bf16)
    Wo3 = Wo.reshape(HQ_LOCAL, Dh, D_MODEL).astype(bf16)

    out = pl.pallas_call(
        _fused_body,
        out_shape=jax.ShapeDtypeStruct((ROWS, D_MODEL), f32),
        in_specs=[pl.BlockSpec(memory_space=pltpu.VMEM)] * 5,
        out_specs=pl.BlockSpec(memory_space=pltpu.VMEM),
        scratch_shapes=[
            pltpu.VMEM((ROWS, D_MODEL), bf16),
            pltpu.VMEM((HQ_LOCAL, ROWS, Dh), bf16),
            pltpu.VMEM((2, CHUNK, D_MODEL), bf16),
            pltpu.VMEM((2, N_STEP, CHUNK, D_MODEL), bf16),
            pltpu.SemaphoreType.DMA((2, N_STEP)),
            pltpu.SemaphoreType.DMA((2, N_STEP)),
        ],
        compiler_params=pltpu.CompilerParams(collective_id=0),
    )(x.reshape(ROWS, D_MODEL), WqT, KT, Vl, Wo3)
    return out.reshape(B, Sq, D_MODEL)


# device time: 44267 ns/iter; 1.2102x vs baseline; 1.1369x over previous
import jax
import jax.numpy as jnp
from jax import lax
from jax.experimental import pallas as pl
from jax.experimental.pallas import tpu as pltpu

N_DEV = 4
B, Sq, Skv, HQ_TOTAL, Dh = 2, 512, 512, 32, 64
HQ_LOCAL = HQ_TOTAL // N_DEV
D_MODEL = 768
BLK = 64
D_LOCAL = HQ_LOCAL * Dh

ROWS = B * Sq
N_CHUNK = 2 * N_DEV
CHUNK = ROWS // N_CHUNK
N_STEP = 2 * (N_DEV - 1)

bf16 = jnp.bfloat16
f32 = jnp.float32


def _m4(e):
    return lax.rem(e, N_DEV)


def _fused_body(
    x_ref, wq_ref, kt_ref, v_ref, wo_ref,
    out_ref,
    xb, qbuf, stage_ref, recv_ref, send_sems, recv_sems,
):
    my = lax.axis_index("i")
    left = _m4(my + N_DEV - 1)
    right = _m4(my + 1)

    xb[...] = x_ref[...].astype(bf16)
    for h in range(HQ_LOCAL):
        qbuf[h] = (
            jnp.dot(xb[...], wq_ref[h], preferred_element_type=f32) * 0.125
        ).astype(bf16)

    qb = lax.broadcasted_iota(jnp.int32, (Sq, Skv), 0) // BLK
    kb = lax.broadcasted_iota(jnp.int32, (Sq, Skv), 1) // BLK
    mask = (qb == kb) | (kb == 0) | (lax.rem(qb + kb, 3) == 0)
    bias = jnp.where(mask, 0.0, -1e9).astype(f32)

    for b in range(B):
        ctxs = []
        for h in range(HQ_LOCAL):
            bh = b * HQ_LOCAL + h
            q = qbuf[h, b * Sq:(b + 1) * Sq, :]
            s = jnp.dot(q, kt_ref[bh], preferred_element_type=f32) + bias
            e = jnp.exp(s)
            rs = 1.0 / jnp.sum(e, axis=1, keepdims=True)
            w = (e * rs).astype(bf16)
            ctxs.append(
                jnp.dot(w, v_ref[bh], preferred_element_type=f32).astype(bf16)
            )
        ctx_cat = jnp.concatenate(ctxs, axis=1)
        out_ref[b * Sq:(b + 1) * Sq, :] = jnp.dot(
            ctx_cat, wo_ref[...], preferred_element_type=f32
        )

    barrier_sem = pltpu.get_barrier_semaphore()
    for nbr in (left, right):
        pl.semaphore_signal(
            barrier_sem, inc=1,
            device_id=(nbr,), device_id_type=pl.DeviceIdType.MESH,
        )
    pl.semaphore_wait(barrier_sem, 2)

    def rows(c):
        return pl.ds(c * CHUNK, CHUNK)

    for s in range(N_DEV - 1):
        plan = [
            (0, right, _m4(my + N_DEV - s), _m4(my + N_DEV - s - 1)),
            (1, left, N_DEV + _m4(my + s), N_DEV + _m4(my + s + 1)),
        ]
        rdmas = []
        for d, nbr, c_send, _ in plan:
            stage_ref[d] = out_ref[rows(c_send), :].astype(bf16)
            rdma = pltpu.make_async_remote_copy(
                src_ref=stage_ref.at[d],
                dst_ref=recv_ref.at[d, s],
                send_sem=send_sems.at[d, s],
                recv_sem=recv_sems.at[d, s],
                device_id=(nbr,),
                device_id_type=pl.DeviceIdType.MESH,
            )
            rdma.start()
            rdmas.append(rdma)
        for (d, _, _, c_recv), rdma in zip(plan, rdmas):
            rdma.wait()
            out_ref[rows(c_recv), :] += recv_ref[d, s].astype(f32)

    for s in range(N_DEV - 1):
        t = N_DEV - 1 + s
        plan = [
            (0, right, _m4(my + N_DEV + 1 - s), _m4(my + N_DEV - s)),
            (1, left, N_DEV + _m4(my + N_DEV - 1 + s), N_DEV + _m4(my + s)),
        ]
        rdmas = []
        for d, nbr, c_send, _ in plan:
            stage_ref[d] = out_ref[rows(c_send), :].astype(bf16)
            rdma = pltpu.make_async_remote_copy(
                src_ref=stage_ref.at[d],
                dst_ref=recv_ref.at[d, t],
                send_sem=send_sems.at[d, t],
                recv_sem=recv_sems.at[d, t],
                device_id=(nbr,),
                device_id_type=pl.DeviceIdType.MESH,
            )
            rdma.start()
            rdmas.append(rdma)
        for (d, _, _, c_recv), rdma in zip(plan, rdmas):
            rdma.wait()
            out_ref[rows(c_recv), :] = recv_ref[d, t].astype(f32)


def kernel(x, Wq, K_ext, V_ext, Wo):
    my = lax.axis_index("i")

    Kl = lax.dynamic_slice_in_dim(K_ext, my * HQ_LOCAL, HQ_LOCAL, axis=2)
    KT = Kl.transpose(0, 2, 3, 1).reshape(B * HQ_LOCAL, Dh, Skv).astype(bf16)
    Vl = (
        lax.dynamic_slice_in_dim(V_ext, my * HQ_LOCAL, HQ_LOCAL, axis=2)
        .transpose(0, 2, 1, 3)
        .reshape(B * HQ_LOCAL, Skv, Dh)
        .astype(bf16)
    )
    WqT = Wq.reshape(D_MODEL, HQ_LOCAL, Dh).transpose(1, 0, 2).astype(bf16)
    Wo3 = Wo.astype(bf16)

    out = pl.pallas_call(
        _fused_body,
        out_shape=jax.ShapeDtypeStruct((ROWS, D_MODEL), f32),
        in_specs=[pl.BlockSpec(memory_space=pltpu.VMEM)] * 5,
        out_specs=pl.BlockSpec(memory_space=pltpu.VMEM),
        scratch_shapes=[
            pltpu.VMEM((ROWS, D_MODEL), bf16),
            pltpu.VMEM((HQ_LOCAL, ROWS, Dh), bf16),
            pltpu.VMEM((2, CHUNK, D_MODEL), bf16),
            pltpu.VMEM((2, N_STEP, CHUNK, D_MODEL), bf16),
            pltpu.SemaphoreType.DMA((2, N_STEP)),
            pltpu.SemaphoreType.DMA((2, N_STEP)),
        ],
        compiler_params=pltpu.CompilerParams(collective_id=0),
    )(x.reshape(ROWS, D_MODEL), WqT, KT, Vl, Wo3)
    return out.reshape(B, Sq, D_MODEL)


# device time: 39614 ns/iter; 1.3524x vs baseline; 1.1175x over previous
import jax
import jax.numpy as jnp
from jax import lax
from jax.experimental import pallas as pl
from jax.experimental.pallas import tpu as pltpu

N_DEV = 4
B, Sq, Skv, HQ_TOTAL, Dh = 2, 512, 512, 32, 64
HQ_LOCAL = HQ_TOTAL // N_DEV
D_MODEL = 768
BLK = 64
D_LOCAL = HQ_LOCAL * Dh

ROWS = B * Sq
N_CHUNK = 2 * N_DEV
CHUNK = ROWS // N_CHUNK
N_STEP = 2 * (N_DEV - 1)

bf16 = jnp.bfloat16
f32 = jnp.float32


def _m4(e):
    return lax.rem(e, N_DEV)


def _rows(c):
    return pl.ds(c * CHUNK, CHUNK)


def _fused_body(
    x_ref, wq_ref, kt_ref, v_ref, wo_ref,
    out_ref,
    xb, qbuf, rs_stage, rs_recv, ag_stage, ag_recv,
    rs_send_sems, rs_recv_sems, ag_send_sems, ag_recv_sems,
):
    my = lax.axis_index("i")

    barrier_sem = pltpu.get_barrier_semaphore()
    for k in range(1, N_DEV):
        pl.semaphore_signal(
            barrier_sem, inc=1,
            device_id=(_m4(my + k),), device_id_type=pl.DeviceIdType.MESH,
        )
    pl.semaphore_wait(barrier_sem, N_DEV - 1)

    xb[...] = x_ref[...].astype(bf16)
    for h in range(HQ_LOCAL):
        qbuf[h] = (
            jnp.dot(xb[...], wq_ref[h], preferred_element_type=f32) * 0.125
        ).astype(bf16)

    qb = lax.broadcasted_iota(jnp.int32, (Sq, Skv), 0) // BLK
    kb = lax.broadcasted_iota(jnp.int32, (Sq, Skv), 1) // BLK
    mask = (qb == kb) | (kb == 0) | (lax.rem(qb + kb, 3) == 0)
    bias = jnp.where(mask, 0.0, -1e9).astype(f32)

    rs_rdmas = []
    for b in range(B):
        ctxs = []
        for h in range(HQ_LOCAL):
            bh = b * HQ_LOCAL + h
            q = qbuf[h, b * Sq:(b + 1) * Sq, :]
            s = jnp.dot(q, kt_ref[bh], preferred_element_type=f32) + bias
            e = jnp.exp(s)
            rs = 1.0 / jnp.sum(e, axis=1, keepdims=True)
            w = (e * rs).astype(bf16)
            ctxs.append(
                jnp.dot(w, v_ref[bh], preferred_element_type=f32).astype(bf16)
            )
        ctx_cat = jnp.concatenate(ctxs, axis=1)
        out_ref[b * Sq:(b + 1) * Sq, :] = jnp.dot(
            ctx_cat, wo_ref[...], preferred_element_type=f32
        )
        for k in range(1, N_DEV):
            dst = _m4(my + k)
            idx = b * (N_DEV - 1) + (k - 1)
            rs_stage[idx] = out_ref[_rows(N_DEV * b + dst), :].astype(bf16)
            rdma = pltpu.make_async_remote_copy(
                src_ref=rs_stage.at[idx],
                dst_ref=rs_recv.at[b, N_DEV - 1 - k],
                send_sem=rs_send_sems.at[idx],
                recv_sem=rs_recv_sems.at[b, N_DEV - 1 - k],
                device_id=(dst,),
                device_id_type=pl.DeviceIdType.MESH,
            )
            rdma.start()
            rs_rdmas.append(rdma)

    for half in range(B):
        for j in range(N_DEV - 1):
            recv = pltpu.make_async_remote_copy(
                src_ref=rs_stage.at[0],
                dst_ref=rs_recv.at[half, j],
                send_sem=rs_send_sems.at[0],
                recv_sem=rs_recv_sems.at[half, j],
                device_id=(my,),
                device_id_type=pl.DeviceIdType.MESH,
            )
            recv.wait_recv()
        own = N_DEV * half + my
        out_ref[_rows(own), :] += (
            rs_recv[half, 0].astype(f32)
            + rs_recv[half, 1].astype(f32)
            + rs_recv[half, 2].astype(f32)
        )

    ag_rdmas = []
    for half in range(B):
        ag_stage[half] = out_ref[_rows(N_DEV * half + my), :].astype(bf16)
    for half in range(B):
        for k in range(1, N_DEV):
            dst = _m4(my + k)
            rdma = pltpu.make_async_remote_copy(
                src_ref=ag_stage.at[half],
                dst_ref=ag_recv.at[half, N_DEV - 1 - k],
                send_sem=ag_send_sems.at[half, k - 1],
                recv_sem=ag_recv_sems.at[half, N_DEV - 1 - k],
                device_id=(dst,),
                device_id_type=pl.DeviceIdType.MESH,
            )
            rdma.start()
            ag_rdmas.append(rdma)
    for half in range(B):
        for j in range(N_DEV - 1):
            recv = pltpu.make_async_remote_copy(
                src_ref=ag_stage.at[half],
                dst_ref=ag_recv.at[half, j],
                send_sem=ag_send_sems.at[half, 0],
                recv_sem=ag_recv_sems.at[half, j],
                device_id=(my,),
                device_id_type=pl.DeviceIdType.MESH,
            )
            recv.wait_recv()
            src_chip = _m4(my + j + 1)
            out_ref[_rows(N_DEV * half + src_chip), :] = (
                ag_recv[half, j].astype(f32)
            )

    for rdma in rs_rdmas + ag_rdmas:
        rdma.wait_send()


def kernel(x, Wq, K_ext, V_ext, Wo):
    my = lax.axis_index("i")

    Kl = lax.dynamic_slice_in_dim(K_ext, my * HQ_LOCAL, HQ_LOCAL, axis=2)
    KT = Kl.transpose(0, 2, 3, 1).reshape(B * HQ_LOCAL, Dh, Skv).astype(bf16)
    Vl = (
        lax.dynamic_slice_in_dim(V_ext, my * HQ_LOCAL, HQ_LOCAL, axis=2)
        .transpose(0, 2, 1, 3)
        .reshape(B * HQ_LOCAL, Skv, Dh)
        .astype(bf16)
    )
    WqT = Wq.reshape(D_MODEL, HQ_LOCAL, Dh).transpose(1, 0, 2).astype(bf16)
    Wo3 = Wo.astype(bf16)

    out = pl.pallas_call(
        _fused_body,
        out_shape=jax.ShapeDtypeStruct((ROWS, D_MODEL), f32),
        in_specs=[pl.BlockSpec(memory_space=pltpu.VMEM)] * 5,
        out_specs=pl.BlockSpec(memory_space=pltpu.VMEM),
        scratch_shapes=[
            pltpu.VMEM((ROWS, D_MODEL), bf16),
            pltpu.VMEM((HQ_LOCAL, ROWS, Dh), bf16),
            pltpu.VMEM((B * (N_DEV - 1), CHUNK, D_MODEL), bf16),
            pltpu.VMEM((B, N_DEV - 1, CHUNK, D_MODEL), bf16),
            pltpu.VMEM((B, CHUNK, D_MODEL), bf16),
            pltpu.VMEM((B, N_DEV - 1, CHUNK, D_MODEL), bf16),
            pltpu.SemaphoreType.DMA((B * (N_DEV - 1),)),
            pltpu.SemaphoreType.DMA((B, N_DEV - 1)),
            pltpu.SemaphoreType.DMA((B, N_DEV - 1)),
            pltpu.SemaphoreType.DMA((B, N_DEV - 1)),
        ],
        compiler_params=pltpu.CompilerParams(collective_id=0),
    )(x.reshape(ROWS, D_MODEL), WqT, KT, Vl, Wo3)
    return out.reshape(B, Sq, D_MODEL)


# device time: 37455 ns/iter; 1.4303x vs baseline; 1.0576x over previous
import jax
import jax.numpy as jnp
from jax import lax
from jax.experimental import pallas as pl
from jax.experimental.pallas import tpu as pltpu

N_DEV = 4
B, Sq, Skv, HQ_TOTAL, Dh = 2, 512, 512, 32, 64
HQ_LOCAL = HQ_TOTAL // N_DEV
D_MODEL = 768
BLK = 64
D_LOCAL = HQ_LOCAL * Dh

ROWS = B * Sq
N_CHUNK = 2 * N_DEV
CHUNK = ROWS // N_CHUNK
N_STEP = 2 * (N_DEV - 1)

bf16 = jnp.bfloat16
f32 = jnp.float32


def _m4(e):
    return lax.rem(e, N_DEV)


def _rows(c):
    return pl.ds(c * CHUNK, CHUNK)


def _fused_body(
    x_ref, wq_ref, kt_ref, v_ref, wo_ref,
    out_ref,
    xb, qbuf, ctxbuf, rs_stage, rs_recv, ag_stage, ag_recv,
    rs_send_sems, rs_recv_sems, ag_send_sems, ag_recv_sems,
):
    my = lax.axis_index("i")

    barrier_sem = pltpu.get_barrier_semaphore()
    for k in range(1, N_DEV):
        pl.semaphore_signal(
            barrier_sem, inc=1,
            device_id=(_m4(my + k),), device_id_type=pl.DeviceIdType.MESH,
        )
    pl.semaphore_wait(barrier_sem, N_DEV - 1)

    xb[...] = x_ref[...].astype(bf16)
    for h in range(HQ_LOCAL):
        qbuf[h] = (
            jnp.dot(xb[...], wq_ref[h], preferred_element_type=f32) * 0.125
        ).astype(bf16)

    qb = lax.broadcasted_iota(jnp.int32, (Sq, Skv), 0) // BLK
    kb = lax.broadcasted_iota(jnp.int32, (Sq, Skv), 1) // BLK
    mask = (qb == kb) | (kb == 0) | (lax.rem(qb + kb, 3) == 0)
    bias = jnp.where(mask, 0.0, -1e9).astype(bf16)

    rs_rdmas = []
    for b in range(B):
        ctxs = []
        for h in range(HQ_LOCAL):
            bh = b * HQ_LOCAL + h
            q = qbuf[h, b * Sq:(b + 1) * Sq, :]
            s = jnp.dot(q, kt_ref[bh], preferred_element_type=f32)
            e = jnp.exp(s.astype(bf16) + bias)
            rinv = 1.0 / jnp.sum(e, axis=1, keepdims=True, dtype=f32)
            ctx = jnp.dot(e, v_ref[bh], preferred_element_type=f32)
            ctxs.append((ctx * rinv).astype(bf16))
        ctxbuf[...] = jnp.concatenate(ctxs, axis=1)
        for t in range(N_DEV):
            c = _m4(my + 1 + t)
            part = jnp.dot(
                ctxbuf[pl.ds(c * CHUNK, CHUNK), :],
                wo_ref[...],
                preferred_element_type=f32,
            )
            out_ref[_rows(N_DEV * b + c), :] = part
            if t < N_DEV - 1:
                idx = b * (N_DEV - 1) + t
                rs_stage[idx] = part.astype(bf16)
                rdma = pltpu.make_async_remote_copy(
                    src_ref=rs_stage.at[idx],
                    dst_ref=rs_recv.at[b, t],
                    send_sem=rs_send_sems.at[idx],
                    recv_sem=rs_recv_sems.at[b, t],
                    device_id=(c,),
                    device_id_type=pl.DeviceIdType.MESH,
                )
                rdma.start()
                rs_rdmas.append(rdma)

    ag_rdmas = []
    for half in range(B):
        for j in range(N_DEV - 1):
            recv = pltpu.make_async_remote_copy(
                src_ref=rs_stage.at[0],
                dst_ref=rs_recv.at[half, j],
                send_sem=rs_send_sems.at[0],
                recv_sem=rs_recv_sems.at[half, j],
                device_id=(my,),
                device_id_type=pl.DeviceIdType.MESH,
            )
            recv.wait_recv()
        own = N_DEV * half + my
        out_ref[_rows(own), :] += (
            rs_recv[half, 0].astype(f32)
            + rs_recv[half, 1].astype(f32)
            + rs_recv[half, 2].astype(f32)
        )
        ag_stage[half] = out_ref[_rows(own), :].astype(bf16)
        for k in range(1, N_DEV):
            dst = _m4(my + k)
            rdma = pltpu.make_async_remote_copy(
                src_ref=ag_stage.at[half],
                dst_ref=ag_recv.at[half, N_DEV - 1 - k],
                send_sem=ag_send_sems.at[half, k - 1],
                recv_sem=ag_recv_sems.at[half, N_DEV - 1 - k],
                device_id=(dst,),
                device_id_type=pl.DeviceIdType.MESH,
            )
            rdma.start()
            ag_rdmas.append(rdma)
    for half in range(B):
        for j in range(N_DEV - 1):
            recv = pltpu.make_async_remote_copy(
                src_ref=ag_stage.at[half],
                dst_ref=ag_recv.at[half, j],
                send_sem=ag_send_sems.at[half, 0],
                recv_sem=ag_recv_sems.at[half, j],
                device_id=(my,),
                device_id_type=pl.DeviceIdType.MESH,
            )
            recv.wait_recv()
            src_chip = _m4(my + j + 1)
            out_ref[_rows(N_DEV * half + src_chip), :] = (
                ag_recv[half, j].astype(f32)
            )

    for rdma in rs_rdmas + ag_rdmas:
        rdma.wait_send()


def kernel(x, Wq, K_ext, V_ext, Wo):
    my = lax.axis_index("i")

    Kl = lax.dynamic_slice_in_dim(K_ext, my * HQ_LOCAL, HQ_LOCAL, axis=2)
    KT = Kl.transpose(0, 2, 3, 1).reshape(B * HQ_LOCAL, Dh, Skv).astype(bf16)
    Vl = (
        lax.dynamic_slice_in_dim(V_ext, my * HQ_LOCAL, HQ_LOCAL, axis=2)
        .transpose(0, 2, 1, 3)
        .reshape(B * HQ_LOCAL, Skv, Dh)
        .astype(bf16)
    )
    WqT = Wq.reshape(D_MODEL, HQ_LOCAL, Dh).transpose(1, 0, 2).astype(bf16)
    Wo3 = Wo.astype(bf16)

    out = pl.pallas_call(
        _fused_body,
        out_shape=jax.ShapeDtypeStruct((ROWS, D_MODEL), f32),
        in_specs=[pl.BlockSpec(memory_space=pltpu.VMEM)] * 5,
        out_specs=pl.BlockSpec(memory_space=pltpu.VMEM),
        scratch_shapes=[
            pltpu.VMEM((ROWS, D_MODEL), bf16),
            pltpu.VMEM((HQ_LOCAL, ROWS, Dh), bf16),
            pltpu.VMEM((Sq, D_LOCAL), bf16),
            pltpu.VMEM((B * (N_DEV - 1), CHUNK, D_MODEL), bf16),
            pltpu.VMEM((B, N_DEV - 1, CHUNK, D_MODEL), bf16),
            pltpu.VMEM((B, CHUNK, D_MODEL), bf16),
            pltpu.VMEM((B, N_DEV - 1, CHUNK, D_MODEL), bf16),
            pltpu.SemaphoreType.DMA((B * (N_DEV - 1),)),
            pltpu.SemaphoreType.DMA((B, N_DEV - 1)),
            pltpu.SemaphoreType.DMA((B, N_DEV - 1)),
            pltpu.SemaphoreType.DMA((B, N_DEV - 1)),
        ],
        compiler_params=pltpu.CompilerParams(collective_id=0),
    )(x.reshape(ROWS, D_MODEL), WqT, KT, Vl, Wo3)
    return out.reshape(B, Sq, D_MODEL)
